# Initial kernel scaffold; baseline (speedup 1.0000x reference)
import jax
import jax.numpy as jnp
from jax.experimental import pallas as pl


def kernel(x, edge_index, batch, embed, W1l, b1l, W1r, W2l, b2l, W2r, Wout, bout):
    raise NotImplementedError("write your pallas kernel here")


# speedup vs baseline: 5.4068x; 5.4068x over previous
import functools

import jax
import jax.numpy as jnp
from jax import lax
from jax.experimental import pallas as pl
from jax.experimental.pallas import tpu as pltpu
from jax.experimental.pallas import tpu_sc as plsc

N = 50000
E = 800000
H = 64
V = 1024
G = 512
C = 10

NC = 2
NS = 16
NW = NC * NS

NH = 25088
NPAD = 2 * NH
RPT = NPAD // NW

ACC_ROWS = 26624
TRASH = 25600
BIGDST = 2 * NH + 100

CH = 128
EPT = 50176
EPAD = EPT * NS
NCHUNK = EPT // CH

DR = 3200
EPW = EPAD // NW
DCH = 784

PR = 528

_mesh = plsc.VectorSubcoreMesh(core_axis_name="c", subcore_axis_name="s")
_params = pltpu.CompilerParams(needs_layout_passes=False,
                               use_tc_tiling_on_sc=False)
_f32 = jnp.float32
_i32 = jnp.int32


def _i16(buf, k):
    return buf[pl.ds(k * 16, 16)]


def _make_pre():
    out_type = [jax.ShapeDtypeStruct((NPAD, H), _f32),
                jax.ShapeDtypeStruct((NPAD, H), _f32),
                jax.ShapeDtypeStruct((2 * DR, 16), _f32)]
    scratch = [
        pltpu.VMEM_SHARED((DR, 16), _f32),
        pltpu.VMEM((DR, 16), _f32),
        pltpu.VMEM((DCH,), _i32),
        pltpu.VMEM((DCH,), _i32),
        pltpu.VMEM((112,), _i32),
        pltpu.VMEM((112, H), _f32),
        pltpu.VMEM((200, 16), _f32),
        pltpu.VMEM((25, 128), _i32),
        pltpu.SemaphoreType.DMA,
        pltpu.SemaphoreType.DMA,
        pltpu.SemaphoreType.DMA,
    ]

    def sc_pre_body(eL, eR, xp, dstp, z16, iota25, hL_o, r1_o, degp_o,
             spdeg, degacc, db0, db1, xbufr, growbuf, dbounce, iotab,
             isem0, isem1, gsem):
        c = lax.axis_index("c")
        sid = lax.axis_index("s")
        wid = sid * NC + c
        ones = jnp.ones((16,), _f32)

        for j in range(8):
            pltpu.sync_copy(z16, degacc.at[pl.ds(j * 400, 400)])
        pltpu.sync_copy(z16.at[pl.ds(0, 200)], spdeg.at[pl.ds(sid * 200, 200)])
        pltpu.sync_copy(iota25, iotab)
        plsc.subcore_barrier()

        for t in range(RPT // 112):
            rbase = wid * RPT + t * 112
            pltpu.sync_copy(xp.at[pl.ds(rbase, 112)], xbufr)
            pltpu.async_copy(eL.at[xbufr], growbuf, gsem).wait()
            pltpu.sync_copy(growbuf, hL_o.at[pl.ds(rbase, 112)])
            pltpu.async_copy(eR.at[xbufr], growbuf, gsem).wait()
            pltpu.sync_copy(growbuf, r1_o.at[pl.ds(rbase, 112)])

        def dstart(a, db, sem):
            return pltpu.async_copy(
                dstp.at[pl.ds(wid * EPW + a * DCH, DCH)], db, sem)

        def dwait(a, db, sem):
            pltpu.make_async_copy(
                dstp.at[pl.ds(wid * EPW + a * DCH, DCH)], db, sem).wait()

        def dscat(db):
            for k in range(DCH // 16):
                dv = _i16(db, k)
                plsc.addupdate_scatter(
                    degacc,
                    [lax.shift_right_logical(dv, 4), lax.bitwise_and(dv, 15)],
                    ones)

        dstart(0, db0, isem0)

        def dbody(t, carry):
            a = 2 * t
            dstart(a + 1, db1, isem1)
            dwait(a, db0, isem0)
            dscat(db0)

            @pl.when(t < EPW // DCH // 2 - 1)
            def _():
                dstart(a + 2, db0, isem0)

            dwait(a + 1, db1, isem1)
            dscat(db1)
            return carry

        lax.fori_loop(0, EPW // DCH // 2, dbody, 0)

        for j in range(25):
            pltpu.sync_copy(degacc.at[pl.ds(j * 128, 128)],
                            spdeg.at[iotab.at[j]], add=True)
        plsc.subcore_barrier()

        pltpu.sync_copy(spdeg.at[pl.ds(sid * 200, 200)], dbounce)
        pltpu.sync_copy(dbounce, degp_o.at[pl.ds(c * DR + sid * 200, 200)])

    return functools.partial(
        pl.kernel, mesh=_mesh, out_type=out_type,
        compiler_params=_params, scratch_types=scratch)(sc_pre_body)


def _make_agg():
    out_type = jax.ShapeDtypeStruct((NPAD, H), _f32)
    scratch = [
        pltpu.VMEM_SHARED((ACC_ROWS, H), _f32),
        pltpu.VMEM((CH, H), _f32), pltpu.VMEM((CH, H), _f32),
        pltpu.VMEM((CH,), _i32), pltpu.VMEM((CH,), _i32),
        pltpu.VMEM((CH,), _i32), pltpu.VMEM((CH,), _i32),
        pltpu.VMEM((CH,), _i32), pltpu.VMEM((CH,), _i32),
        pltpu.VMEM((CH,), _i32), pltpu.VMEM((CH,), _i32),
        pltpu.VMEM((CH,), _i32), pltpu.VMEM((CH,), _i32),
        pltpu.SemaphoreType.DMA,
        pltpu.SemaphoreType.DMA, pltpu.SemaphoreType.DMA,
    ]

    def sc_agg_body(table, srcp, dstp, z64, agg_o, acc,
             rowbuf0, rowbuf1, sbuf0, sbuf1, sbuf2, sbuf3,
             dbuf0, dbuf1, dbuf2, dbuf3, lbuf0, lbuf1,
             isem, gsem0, gsem1):
        c = lax.axis_index("c")
        sid = lax.axis_index("s")
        chalf = c * NH
        sbufs = [sbuf0, sbuf1, sbuf2, sbuf3]
        dbufs = [dbuf0, dbuf1, dbuf2, dbuf3]
        rowbufs = [rowbuf0, rowbuf1]
        lbufs = [lbuf0, lbuf1]
        gsems = [gsem0, gsem1]

        pltpu.sync_copy(z64, rowbuf0)
        for j in range(13):
            pltpu.sync_copy(rowbuf0,
                            acc.at[pl.ds(sid * 1664 + j * 128, 128)])
        plsc.subcore_barrier()

        def ebase(q):
            return sid * EPT + q * CH

        def idx_start(q, m):
            a = pltpu.async_copy(srcp.at[pl.ds(ebase(q), CH)], sbufs[m], isem)
            b = pltpu.async_copy(dstp.at[pl.ds(ebase(q), CH)], dbufs[m], isem)
            return a, b

        def idx_wait(q, m):
            pltpu.make_async_copy(srcp.at[pl.ds(ebase(q), CH)],
                                  sbufs[m], isem).wait()
            pltpu.make_async_copy(dstp.at[pl.ds(ebase(q), CH)],
                                  dbufs[m], isem).wait()

        def rg_start(m, r):
            pltpu.async_copy(table.at[sbufs[m]], rowbufs[r], gsems[r])

        def rg_wait(m, r):
            pltpu.make_async_copy(table.at[sbufs[m]], rowbufs[r],
                                  gsems[r]).wait()

        def scat(m, r):
            lb = lbufs[r]
            for k in range(CH // 16):
                dv = _i16(dbufs[m], k) - chalf
                inr = (dv >= 0) & (dv < NH)
                lb[pl.ds(k * 16, 16)] = jnp.where(inr, dv, TRASH)
            pltpu.sync_copy(rowbufs[r], acc.at[lb], add=True)

        idx_start(0, 0)
        idx_wait(0, 0)
        rg_start(0, 0)

        def block(t, carry):
            for k in range(4):
                q = 4 * t + k
                m, r = k, k % 2
                mn, rn = (k + 1) % 4, (k + 1) % 2

                @pl.when(q + 1 < NCHUNK)
                def _():
                    idx_wait(q + 1, mn)
                    rg_start(mn, rn)

                @pl.when(q + 2 < NCHUNK)
                def _():
                    idx_start(q + 2, (k + 2) % 4)

                rg_wait(m, r)
                scat(m, r)
            return carry

        idx_start(1, 1)
        lax.fori_loop(0, NCHUNK // 4, block, 0)
        plsc.subcore_barrier()

        for t in range(RPT // 112):
            rbase = sid * RPT + t * 112
            pltpu.sync_copy(acc.at[pl.ds(rbase, 112)],
                            rowbuf0.at[pl.ds(0, 112)])
            pltpu.sync_copy(rowbuf0.at[pl.ds(0, 112)],
                            agg_o.at[pl.ds(chalf + rbase, 112)])

    return functools.partial(
        pl.kernel, mesh=_mesh, out_type=out_type,
        compiler_params=_params, scratch_types=scratch)(sc_agg_body)


def _make_pool():
    out_type = [jax.ShapeDtypeStruct((2 * PR, H), _f32),
                jax.ShapeDtypeStruct((NW, 33, 16), _f32)]
    scratch = [
        pltpu.VMEM_SHARED((PR, H), _f32),
        pltpu.VMEM((112, H), _f32),
        pltpu.VMEM((112,), _i32),
        pltpu.VMEM((33, 16), _f32),
    ]

    def sc_pool_body(h2, bp, z64, z16, psum_o, pcnt_o, pacc, rbuf, bbuf,
                     cntloc):
        c = lax.axis_index("c")
        sid = lax.axis_index("s")
        wid = sid * NC + c
        ones = jnp.ones((16,), _f32)

        pltpu.sync_copy(z64, pacc.at[pl.ds(sid * 33, 33)])
        pltpu.sync_copy(z16, cntloc)
        plsc.subcore_barrier()

        for t in range(RPT // 112):
            rbase = wid * RPT + t * 112
            pltpu.sync_copy(h2.at[pl.ds(rbase, 112)], rbuf)
            pltpu.sync_copy(bp.at[pl.ds(rbase, 112)], bbuf)
            for k in range(7):
                bv = _i16(bbuf, k)
                plsc.addupdate_scatter(
                    cntloc,
                    [lax.shift_right_logical(bv, 4), lax.bitwise_and(bv, 15)],
                    ones)
            pltpu.sync_copy(rbuf, pacc.at[bbuf], add=True)
        pltpu.sync_copy(cntloc, pcnt_o.at[wid])
        plsc.subcore_barrier()

        pltpu.sync_copy(pacc.at[pl.ds(sid * 33, 33)], rbuf.at[pl.ds(0, 33)])
        pltpu.sync_copy(rbuf.at[pl.ds(0, 33)],
                        psum_o.at[pl.ds(c * PR + sid * 33, 33)])

    return functools.partial(
        pl.kernel, mesh=_mesh, out_type=out_type,
        compiler_params=_params, scratch_types=scratch)(sc_pool_body)


_pre = _make_pre()
_agg = _make_agg()
_pool = _make_pool()


def _tc0_body(embed, W1l, W1r, b1l, eL_o, eR_o):
    dn = (((1,), (1,)), ((), ()))
    e = embed[...]
    eL_o[...] = lax.dot_general(e, W1l[...], dn,
                                preferred_element_type=_f32)
    eR_o[...] = lax.dot_general(e, W1r[...], dn,
                                preferred_element_type=_f32) + b1l[...]


def _tc0(embed, W1l, W1r, b1l):
    return pl.pallas_call(
        _tc0_body,
        out_shape=[jax.ShapeDtypeStruct((V, H), _f32),
                   jax.ShapeDtypeStruct((V, H), _f32)],
    )(embed, W1l, W1r, b1l)


_BLK = 512
_NBLK = NPAD // _BLK


def _tc1_body(agg, r1, deg, h1_o):
    recip = 1.0 / jnp.maximum(deg[...], 1.0)
    h1_o[...] = jnp.maximum(agg[...] * recip + r1[...], 0.0)


def _tc1(agg, r1, deg):
    bs = pl.BlockSpec((_BLK, H), lambda i: (i, 0))
    bd = pl.BlockSpec((_BLK, 1), lambda i: (i, 0))
    return pl.pallas_call(
        _tc1_body, grid=(_NBLK,),
        in_specs=[bs, bs, bd], out_specs=bs,
        out_shape=jax.ShapeDtypeStruct((NPAD, H), _f32),
    )(agg, r1, deg)


def _tc2_body(agg, deg, h1, W2l, b2l, W2r, h2_o):
    dn = (((1,), (1,)), ((), ()))
    mean = agg[...] * (1.0 / jnp.maximum(deg[...], 1.0))
    z = (lax.dot_general(mean, W2l[...], dn, preferred_element_type=_f32)
         + b2l[...]
         + lax.dot_general(h1[...], W2r[...], dn,
                           preferred_element_type=_f32))
    h2_o[...] = jnp.maximum(z, 0.0)


def _tc2(agg, deg, h1, W2l, b2l, W2r):
    bs = pl.BlockSpec((_BLK, H), lambda i: (i, 0))
    bd = pl.BlockSpec((_BLK, 1), lambda i: (i, 0))
    bw = pl.BlockSpec((H, H), lambda i: (0, 0))
    bb = pl.BlockSpec((1, H), lambda i: (0, 0))
    return pl.pallas_call(
        _tc2_body, grid=(_NBLK,),
        in_specs=[bs, bd, bs, bw, bb, bw], out_specs=bs,
        out_shape=jax.ShapeDtypeStruct((NPAD, H), _f32),
    )(agg, deg, h1, W2l, b2l, W2r)


def _tc3_body(psum, pcnt, Wout, bout, out_o):
    dn = (((1,), (1,)), ((), ()))
    tot = psum[pl.ds(0, G), :] + psum[pl.ds(PR, G), :]
    cnt = jnp.sum(pcnt[pl.ds(0, G), :], axis=1, keepdims=True)
    pooled = tot * (1.0 / jnp.maximum(cnt, 1.0))
    out_o[...] = (lax.dot_general(pooled, Wout[...], dn,
                                  preferred_element_type=_f32) + bout[...])


def _tc3(psum, pcnt, Wout, bout):
    return pl.pallas_call(
        _tc3_body,
        out_shape=jax.ShapeDtypeStruct((G, C), _f32),
    )(psum, pcnt, Wout, bout)


def kernel(x, edge_index, batch, embed, W1l, b1l, W1r, W2l, b2l, W2r,
           Wout, bout):
    x = x.astype(_i32)
    src = edge_index[0].astype(_i32)
    dst = edge_index[1].astype(_i32)
    batch = batch.astype(_i32)

    xp = jnp.concatenate([x, jnp.zeros((NPAD - N,), _i32)])
    srcp = jnp.concatenate([src, jnp.zeros((EPAD - E,), _i32)])
    dstp = jnp.concatenate([dst, jnp.full((EPAD - E,), BIGDST, _i32)])
    bp = jnp.concatenate([batch, jnp.full((NPAD - N,), G, _i32)])
    z64 = jnp.zeros((CH, H), _f32)
    zd16 = jnp.zeros((400, 16), _f32)
    zp64 = jnp.zeros((33, H), _f32)
    zp16 = jnp.zeros((33, 16), _f32)
    iota25 = jnp.arange(25 * 128, dtype=_i32).reshape(25, 128)

    eL, eR = _tc0(embed, W1l, W1r, b1l.reshape(1, H))

    hL, r1, degp = _pre(eL, eR, xp, dstp, zd16, iota25)
    dflat = degp.reshape(2, DR * 16)
    deg = jnp.concatenate(
        [(dflat[0] + dflat[1])[:N], jnp.zeros((NPAD - N,), _f32)]
    ).reshape(NPAD, 1)

    agg1 = _agg(hL, srcp, dstp, z64)
    h1 = _tc1(agg1, r1, deg)
    agg2 = _agg(h1, srcp, dstp, z64)
    h2 = _tc2(agg2, deg, h1, W2l, b2l.reshape(1, H), W2r)

    psum, pcnt = _pool(h2, bp, zp64, zp16)
    pcnt_t = pcnt.reshape(NW, PR).T

    return _tc3(psum, pcnt_t, Wout, bout.reshape(1, C))


# speedup vs baseline: 6.2949x; 1.1643x over previous
import functools

import jax
import jax.numpy as jnp
from jax import lax
from jax.experimental import pallas as pl
from jax.experimental.pallas import tpu as pltpu
from jax.experimental.pallas import tpu_sc as plsc

N = 50000
E = 800000
H = 64
V = 1024
G = 512
C = 10

NC = 2
NS = 16
NW = NC * NS

NH = 25088
NPAD = 2 * NH
RPT = NPAD // NW

ACC_ROWS = 26624
TRASH = 25600
BIGDST = 2 * NH + 100

CH = 128
EPT = 50176
EPAD = EPT * NS
NCHUNK = EPT // CH

DR = 3200
EPW = EPAD // NW
DCH = 784

PR = 528

_mesh = plsc.VectorSubcoreMesh(core_axis_name="c", subcore_axis_name="s")
_params = pltpu.CompilerParams(needs_layout_passes=False,
                               use_tc_tiling_on_sc=False)
_f32 = jnp.float32
_i32 = jnp.int32


def _i16(buf, k):
    return buf[pl.ds(k * 16, 16)]


def _make_pre():
    out_type = [jax.ShapeDtypeStruct((NPAD, H), _f32),
                jax.ShapeDtypeStruct((NPAD, H), _f32),
                jax.ShapeDtypeStruct((2 * DR, 16), _f32)]
    scratch = [
        pltpu.VMEM_SHARED((DR, 16), _f32),
        pltpu.VMEM((DR, 16), _f32),
        pltpu.VMEM((DCH,), _i32),
        pltpu.VMEM((DCH,), _i32),
        pltpu.VMEM((112,), _i32),
        pltpu.VMEM((112, H), _f32),
        pltpu.VMEM((200, 16), _f32),
        pltpu.VMEM((25, 128), _i32),
        pltpu.SemaphoreType.DMA,
        pltpu.SemaphoreType.DMA,
        pltpu.SemaphoreType.DMA,
    ]

    def sc_pre_body(eL, eR, xp, dstp, z16, iota25, hL_o, r1_o, degp_o,
             spdeg, degacc, db0, db1, xbufr, growbuf, dbounce, iotab,
             isem0, isem1, gsem):
        c = lax.axis_index("c")
        sid = lax.axis_index("s")
        wid = sid * NC + c
        ones = jnp.ones((16,), _f32)

        for j in range(8):
            pltpu.sync_copy(z16, degacc.at[pl.ds(j * 400, 400)])
        pltpu.sync_copy(z16.at[pl.ds(0, 200)], spdeg.at[pl.ds(sid * 200, 200)])
        pltpu.sync_copy(iota25, iotab)
        plsc.subcore_barrier()

        for t in range(RPT // 112):
            rbase = wid * RPT + t * 112
            pltpu.sync_copy(xp.at[pl.ds(rbase, 112)], xbufr)
            pltpu.async_copy(eL.at[xbufr], growbuf, gsem).wait()
            pltpu.sync_copy(growbuf, hL_o.at[pl.ds(rbase, 112)])
            pltpu.async_copy(eR.at[xbufr], growbuf, gsem).wait()
            pltpu.sync_copy(growbuf, r1_o.at[pl.ds(rbase, 112)])

        def dstart(a, db, sem):
            return pltpu.async_copy(
                dstp.at[pl.ds(wid * EPW + a * DCH, DCH)], db, sem)

        def dwait(a, db, sem):
            pltpu.make_async_copy(
                dstp.at[pl.ds(wid * EPW + a * DCH, DCH)], db, sem).wait()

        def dscat(db):
            for k in range(DCH // 16):
                dv = _i16(db, k)
                plsc.addupdate_scatter(
                    degacc,
                    [lax.shift_right_logical(dv, 4), lax.bitwise_and(dv, 15)],
                    ones)

        dstart(0, db0, isem0)

        def dbody(t, carry):
            a = 2 * t
            dstart(a + 1, db1, isem1)
            dwait(a, db0, isem0)
            dscat(db0)

            @pl.when(t < EPW // DCH // 2 - 1)
            def _():
                dstart(a + 2, db0, isem0)

            dwait(a + 1, db1, isem1)
            dscat(db1)
            return carry

        lax.fori_loop(0, EPW // DCH // 2, dbody, 0)

        for j in range(25):
            pltpu.sync_copy(degacc.at[pl.ds(j * 128, 128)],
                            spdeg.at[iotab.at[j]], add=True)
        plsc.subcore_barrier()

        pltpu.sync_copy(spdeg.at[pl.ds(sid * 200, 200)], dbounce)
        pltpu.sync_copy(dbounce, degp_o.at[pl.ds(c * DR + sid * 200, 200)])

    return functools.partial(
        pl.kernel, mesh=_mesh, out_type=out_type,
        compiler_params=_params, scratch_types=scratch)(sc_pre_body)


WCAP = EPW + 512
PCH = 784
PSTEPS = PCH // 16

def _make_part():
    out_type = [jax.ShapeDtypeStruct((NW * 2 * WCAP,), _i32),
                jax.ShapeDtypeStruct((NW * 2 * WCAP,), _i32),
                jax.ShapeDtypeStruct((NW, 16), _i32)]
    scratch = [
        pltpu.VMEM((PCH,), _i32), pltpu.VMEM((PCH,), _i32),
        pltpu.VMEM((PCH,), _i32), pltpu.VMEM((PCH,), _i32),
        pltpu.VMEM((544,), _i32), pltpu.VMEM((544,), _i32),
        pltpu.VMEM((544,), _i32), pltpu.VMEM((544,), _i32),
        pltpu.VMEM((16,), _i32),
        pltpu.SemaphoreType.DMA, pltpu.SemaphoreType.DMA,
    ]

    def sc_part_body(srcp, dstp, psrc_o, pdst_o, cnt_o,
                     sb0, sb1, db0, db1, osrc0, odst0, osrc1, odst1,
                     cbuf, isem0, isem1):
        c = lax.axis_index("c")
        sid = lax.axis_index("s")
        wid = sid * NC + c
        iota16 = lax.iota(_i32, 16)
        trash16 = jnp.full((16,), TRASH, _i32)
        zero16 = jnp.zeros((16,), _i32)
        obufs = [(osrc0, odst0), (osrc1, odst1)]
        rbase = [(wid * 2 + 0) * WCAP, (wid * 2 + 1) * WCAP]

        def ld_start(a, sb, db, sem):
            pltpu.async_copy(
                srcp.at[pl.ds(wid * EPW + a * PCH, PCH)], sb, sem)
            return pltpu.async_copy(
                dstp.at[pl.ds(wid * EPW + a * PCH, PCH)], db, sem)

        def ld_wait(a, sb, db, sem):
            pltpu.make_async_copy(
                srcp.at[pl.ds(wid * EPW + a * PCH, PCH)], sb, sem).wait()
            pltpu.make_async_copy(
                dstp.at[pl.ds(wid * EPW + a * PCH, PCH)], db, sem).wait()

        def half_step(h, sv, dv, bufcnt, gblk):
            osrc, odst = obufs[h]
            if h == 0:
                mask = dv < NH
                dloc = dv
            else:
                mask = (dv >= NH) & (dv < 2 * NH)
                dloc = dv - NH
            m32 = jnp.where(mask, 1, 0)
            pos = plsc.cumsum(m32)
            idx = bufcnt + pos - 1
            plsc.store_scatter(odst, [idx], dloc, mask=mask)
            plsc.store_scatter(osrc, [idx], sv, mask=mask)
            bufcnt = bufcnt + jnp.sum(m32)
            flush = bufcnt >= 512
            goff = rbase[h] + gblk * 512

            @pl.when(flush)
            def _():
                pltpu.sync_copy(osrc.at[pl.ds(0, 512)],
                                psrc_o.at[pl.ds(goff, 512)])
                pltpu.sync_copy(odst.at[pl.ds(0, 512)],
                                pdst_o.at[pl.ds(goff, 512)])
                osrc[pl.ds(0, 16)] = osrc[pl.ds(512, 16)]
                odst[pl.ds(0, 16)] = odst[pl.ds(512, 16)]

            step = jnp.where(flush, 512, 0)
            binc = jnp.where(flush, 1, 0)
            return bufcnt - step, gblk + binc

        def chunk(sb, db, carry):
            def stepf(k, carry):
                b0, g0, b1, g1 = carry
                kidx = k * 16 + iota16
                sv = plsc.load_gather(sb, [kidx])
                dv = plsc.load_gather(db, [kidx])
                b0, g0 = half_step(0, sv, dv, b0, g0)
                b1, g1 = half_step(1, sv, dv, b1, g1)
                return (b0, g0, b1, g1)

            return lax.fori_loop(0, PSTEPS, stepf, carry)

        ld_start(0, sb0, db0, isem0)
        ld_start(1, sb1, db1, isem1)

        def dbody(t, carry):
            a = 2 * t
            ld_wait(a, sb0, db0, isem0)
            carry = chunk(sb0, db0, carry)

            @pl.when(t < EPW // PCH // 2 - 1)
            def _():
                ld_start(a + 2, sb0, db0, isem0)

            ld_wait(a + 1, sb1, db1, isem1)
            carry = chunk(sb1, db1, carry)

            @pl.when(t < EPW // PCH // 2 - 1)
            def _():
                ld_start(a + 3, sb1, db1, isem1)

            return carry

        zi = jnp.zeros((), _i32)
        b0, g0, b1, g1 = lax.fori_loop(0, EPW // PCH // 2, dbody,
                                       (zi, zi, zi, zi))

        def finish(h, bufcnt, gblk):
            osrc, odst = obufs[h]
            for j in range(32):
                idx = j * 16 + iota16
                mask = idx >= bufcnt
                plsc.store_scatter(odst, [idx], trash16, mask=mask)
                plsc.store_scatter(osrc, [idx], zero16, mask=mask)
            goff = rbase[h] + gblk * 512
            pltpu.sync_copy(osrc.at[pl.ds(0, 512)],
                            psrc_o.at[pl.ds(goff, 512)])
            pltpu.sync_copy(odst.at[pl.ds(0, 512)],
                            pdst_o.at[pl.ds(goff, 512)])
            return (gblk + 1) * 512

        tot0 = finish(0, b0, g0)
        tot1 = finish(1, b1, g1)

        cbuf[...] = (jnp.where(iota16 == 0, tot0, 0)
                     + jnp.where(iota16 == 1, tot1, 0))
        pltpu.sync_copy(cbuf, cnt_o.at[wid])

    return functools.partial(
        pl.kernel, mesh=_mesh, out_type=out_type,
        compiler_params=_params, scratch_types=scratch)(sc_part_body)


def _make_agg():
    out_type = jax.ShapeDtypeStruct((NPAD, H), _f32)
    scratch = [
        pltpu.VMEM_SHARED((ACC_ROWS, H), _f32),
        pltpu.VMEM((CH, H), _f32), pltpu.VMEM((CH, H), _f32),
        pltpu.VMEM((CH,), _i32), pltpu.VMEM((CH,), _i32),
        pltpu.VMEM((CH,), _i32), pltpu.VMEM((CH,), _i32),
        pltpu.VMEM((CH,), _i32), pltpu.VMEM((CH,), _i32),
        pltpu.VMEM((CH,), _i32), pltpu.VMEM((CH,), _i32),
        pltpu.VMEM((16,), _i32),
        pltpu.SemaphoreType.DMA,
        pltpu.SemaphoreType.DMA, pltpu.SemaphoreType.DMA,
    ]

    def sc_agg_body(table, psrc, pdst, cntp, z64, agg_o, acc,
             rowbuf0, rowbuf1, sbuf0, sbuf1, sbuf2, sbuf3,
             dbuf0, dbuf1, dbuf2, dbuf3, cbuf,
             isem, gsem0, gsem1):
        c = lax.axis_index("c")
        sid = lax.axis_index("s")
        chalf = c * NH
        iota16 = lax.iota(_i32, 16)
        sbufs = [sbuf0, sbuf1, sbuf2, sbuf3]
        dbufs = [dbuf0, dbuf1, dbuf2, dbuf3]
        rowbufs = [rowbuf0, rowbuf1]
        gsems = [gsem0, gsem1]

        pltpu.sync_copy(z64, rowbuf0)
        for j in range(13):
            pltpu.sync_copy(rowbuf0,
                            acc.at[pl.ds(sid * 1664 + j * 128, 128)])
        plsc.subcore_barrier()

        def do_region(w):
            pltpu.sync_copy(cntp.at[w], cbuf)
            cntw = jnp.sum(jnp.where(iota16 == c, cbuf[...], 0))
            nchunk = lax.shift_right_logical(cntw, 7)
            nblk = lax.shift_right_logical(cntw, 9)
            base = (w * 2 + c) * WCAP

            def ebase(q):
                return base + q * CH

            def idx_start(q, m):
                pltpu.async_copy(psrc.at[pl.ds(ebase(q), CH)], sbufs[m], isem)
                pltpu.async_copy(pdst.at[pl.ds(ebase(q), CH)], dbufs[m], isem)

            def idx_wait(q, m):
                pltpu.make_async_copy(psrc.at[pl.ds(ebase(q), CH)],
                                      sbufs[m], isem).wait()
                pltpu.make_async_copy(pdst.at[pl.ds(ebase(q), CH)],
                                      dbufs[m], isem).wait()

            def rg_start(m, r):
                pltpu.async_copy(table.at[sbufs[m]], rowbufs[r], gsems[r])

            def rg_wait(m, r):
                pltpu.make_async_copy(table.at[sbufs[m]], rowbufs[r],
                                      gsems[r]).wait()

            def scat(m, r):
                pltpu.sync_copy(rowbufs[r], acc.at[dbufs[m]], add=True)

            idx_start(0, 0)
            idx_wait(0, 0)
            rg_start(0, 0)
            idx_start(1, 1)

            def block(t, carry):
                for k in range(4):
                    q = 4 * t + k
                    m, r = k, k % 2
                    mn, rn = (k + 1) % 4, (k + 1) % 2

                    @pl.when(q + 1 < nchunk)
                    def _():
                        idx_wait(q + 1, mn)
                        rg_start(mn, rn)

                    @pl.when(q + 2 < nchunk)
                    def _():
                        idx_start(q + 2, (k + 2) % 4)

                    rg_wait(m, r)
                    scat(m, r)
                return carry

            lax.fori_loop(0, nblk, block, 0)

        do_region(2 * sid)
        do_region(2 * sid + 1)
        plsc.subcore_barrier()

        for t in range(RPT // 112):
            rbase = sid * RPT + t * 112
            pltpu.sync_copy(acc.at[pl.ds(rbase, 112)],
                            rowbuf0.at[pl.ds(0, 112)])
            pltpu.sync_copy(rowbuf0.at[pl.ds(0, 112)],
                            agg_o.at[pl.ds(chalf + rbase, 112)])

    return functools.partial(
        pl.kernel, mesh=_mesh, out_type=out_type,
        compiler_params=_params, scratch_types=scratch)(sc_agg_body)


def _make_pool():
    out_type = [jax.ShapeDtypeStruct((2 * PR, H), _f32),
                jax.ShapeDtypeStruct((NW, 33, 16), _f32)]
    scratch = [
        pltpu.VMEM_SHARED((PR, H), _f32),
        pltpu.VMEM((112, H), _f32),
        pltpu.VMEM((112,), _i32),
        pltpu.VMEM((33, 16), _f32),
    ]

    def sc_pool_body(h2, bp, z64, z16, psum_o, pcnt_o, pacc, rbuf, bbuf,
                     cntloc):
        c = lax.axis_index("c")
        sid = lax.axis_index("s")
        wid = sid * NC + c
        ones = jnp.ones((16,), _f32)

        pltpu.sync_copy(z64, pacc.at[pl.ds(sid * 33, 33)])
        pltpu.sync_copy(z16, cntloc)
        plsc.subcore_barrier()

        for t in range(RPT // 112):
            rbase = wid * RPT + t * 112
            pltpu.sync_copy(h2.at[pl.ds(rbase, 112)], rbuf)
            pltpu.sync_copy(bp.at[pl.ds(rbase, 112)], bbuf)
            for k in range(7):
                bv = _i16(bbuf, k)
                plsc.addupdate_scatter(
                    cntloc,
                    [lax.shift_right_logical(bv, 4), lax.bitwise_and(bv, 15)],
                    ones)
            pltpu.sync_copy(rbuf, pacc.at[bbuf], add=True)
        pltpu.sync_copy(cntloc, pcnt_o.at[wid])
        plsc.subcore_barrier()

        pltpu.sync_copy(pacc.at[pl.ds(sid * 33, 33)], rbuf.at[pl.ds(0, 33)])
        pltpu.sync_copy(rbuf.at[pl.ds(0, 33)],
                        psum_o.at[pl.ds(c * PR + sid * 33, 33)])

    return functools.partial(
        pl.kernel, mesh=_mesh, out_type=out_type,
        compiler_params=_params, scratch_types=scratch)(sc_pool_body)


_pre = _make_pre()
_part = _make_part()
_agg = _make_agg()
_pool = _make_pool()


def _tc0_body(embed, W1l, W1r, b1l, eL_o, eR_o):
    dn = (((1,), (1,)), ((), ()))
    e = embed[...]
    eL_o[...] = lax.dot_general(e, W1l[...], dn,
                                preferred_element_type=_f32)
    eR_o[...] = lax.dot_general(e, W1r[...], dn,
                                preferred_element_type=_f32) + b1l[...]


def _tc0(embed, W1l, W1r, b1l):
    return pl.pallas_call(
        _tc0_body,
        out_shape=[jax.ShapeDtypeStruct((V, H), _f32),
                   jax.ShapeDtypeStruct((V, H), _f32)],
    )(embed, W1l, W1r, b1l)


_BLK = 512
_NBLK = NPAD // _BLK


def _tc1_body(agg, r1, deg, h1_o):
    recip = 1.0 / jnp.maximum(deg[...], 1.0)
    h1_o[...] = jnp.maximum(agg[...] * recip + r1[...], 0.0)


def _tc1(agg, r1, deg):
    bs = pl.BlockSpec((_BLK, H), lambda i: (i, 0))
    bd = pl.BlockSpec((_BLK, 1), lambda i: (i, 0))
    return pl.pallas_call(
        _tc1_body, grid=(_NBLK,),
        in_specs=[bs, bs, bd], out_specs=bs,
        out_shape=jax.ShapeDtypeStruct((NPAD, H), _f32),
    )(agg, r1, deg)


def _tc2_body(agg, deg, h1, W2l, b2l, W2r, h2_o):
    dn = (((1,), (1,)), ((), ()))
    mean = agg[...] * (1.0 / jnp.maximum(deg[...], 1.0))
    z = (lax.dot_general(mean, W2l[...], dn, preferred_element_type=_f32)
         + b2l[...]
         + lax.dot_general(h1[...], W2r[...], dn,
                           preferred_element_type=_f32))
    h2_o[...] = jnp.maximum(z, 0.0)


def _tc2(agg, deg, h1, W2l, b2l, W2r):
    bs = pl.BlockSpec((_BLK, H), lambda i: (i, 0))
    bd = pl.BlockSpec((_BLK, 1), lambda i: (i, 0))
    bw = pl.BlockSpec((H, H), lambda i: (0, 0))
    bb = pl.BlockSpec((1, H), lambda i: (0, 0))
    return pl.pallas_call(
        _tc2_body, grid=(_NBLK,),
        in_specs=[bs, bd, bs, bw, bb, bw], out_specs=bs,
        out_shape=jax.ShapeDtypeStruct((NPAD, H), _f32),
    )(agg, deg, h1, W2l, b2l, W2r)


def _tc3_body(psum, pcnt, Wout, bout, out_o):
    dn = (((1,), (1,)), ((), ()))
    tot = psum[pl.ds(0, G), :] + psum[pl.ds(PR, G), :]
    cnt = jnp.sum(pcnt[pl.ds(0, G), :], axis=1, keepdims=True)
    pooled = tot * (1.0 / jnp.maximum(cnt, 1.0))
    out_o[...] = (lax.dot_general(pooled, Wout[...], dn,
                                  preferred_element_type=_f32) + bout[...])


def _tc3(psum, pcnt, Wout, bout):
    return pl.pallas_call(
        _tc3_body,
        out_shape=jax.ShapeDtypeStruct((G, C), _f32),
    )(psum, pcnt, Wout, bout)


def kernel(x, edge_index, batch, embed, W1l, b1l, W1r, W2l, b2l, W2r,
           Wout, bout):
    x = x.astype(_i32)
    src = edge_index[0].astype(_i32)
    dst = edge_index[1].astype(_i32)
    batch = batch.astype(_i32)

    xp = jnp.concatenate([x, jnp.zeros((NPAD - N,), _i32)])
    srcp = jnp.concatenate([src, jnp.zeros((EPAD - E,), _i32)])
    dstp = jnp.concatenate([dst, jnp.full((EPAD - E,), BIGDST, _i32)])
    bp = jnp.concatenate([batch, jnp.full((NPAD - N,), G, _i32)])
    z64 = jnp.zeros((CH, H), _f32)
    zd16 = jnp.zeros((400, 16), _f32)
    zp64 = jnp.zeros((33, H), _f32)
    zp16 = jnp.zeros((33, 16), _f32)
    iota25 = jnp.arange(25 * 128, dtype=_i32).reshape(25, 128)

    eL, eR = _tc0(embed, W1l, W1r, b1l.reshape(1, H))

    psrc, pdst, cntp = _part(srcp, dstp)
    hL, r1, degp = _pre(eL, eR, xp, dstp, zd16, iota25)
    dflat = degp.reshape(2, DR * 16)
    deg = jnp.concatenate(
        [(dflat[0] + dflat[1])[:N], jnp.zeros((NPAD - N,), _f32)]
    ).reshape(NPAD, 1)

    agg1 = _agg(hL, psrc, pdst, cntp, z64)
    h1 = _tc1(agg1, r1, deg)
    agg2 = _agg(h1, psrc, pdst, cntp, z64)
    h2 = _tc2(agg2, deg, h1, W2l, b2l.reshape(1, H), W2r)

    psum, pcnt = _pool(h2, bp, zp64, zp16)
    pcnt_t = pcnt.reshape(NW, PR).T

    return _tc3(psum, pcnt_t, Wout, bout.reshape(1, C))
